# Initial kernel scaffold; baseline (speedup 1.0000x reference)
import jax
import jax.numpy as jnp
from jax.experimental import pallas as pl


def kernel(feat_map0, feat_map1, feat_map2, feat_map3, W1, b1, W2, b2):
    raise NotImplementedError("write your pallas kernel here")


# speedup vs baseline: 8.8618x; 8.8618x over previous
import functools

import jax
import jax.numpy as jnp
from jax.experimental import pallas as pl

_C = 256
_HID = 1024


def _mlp_kernel(x_ref, w1_ref, b1_ref, w2_ref, b2_ref, o_ref):
    x = x_ref[0]
    h = jax.lax.dot_general(
        w1_ref[...], x, (((0,), (0,)), ((), ())),
        preferred_element_type=jnp.float32,
    )
    h = jnp.maximum(h + b1_ref[...], 0.0)
    o = jax.lax.dot_general(
        w2_ref[...], h, (((0,), (0,)), ((), ())),
        preferred_element_type=jnp.float32,
    )
    o_ref[0] = o + b2_ref[...]


@functools.partial(jax.jit, static_argnames=("block_n", "interpret"))
def _mlp_map(feat, w1, b1c, w2, b2c, *, block_n, interpret=False):
    b, c, h, w = feat.shape
    n = h * w
    x = feat.reshape(b, c, n)
    bn = min(block_n, n)
    grid = (b, n // bn)
    out = pl.pallas_call(
        _mlp_kernel,
        grid=grid,
        in_specs=[
            pl.BlockSpec((1, c, bn), lambda i, j: (i, 0, j)),
            pl.BlockSpec((_C, _HID), lambda i, j: (0, 0)),
            pl.BlockSpec((_HID, 1), lambda i, j: (0, 0)),
            pl.BlockSpec((_HID, _C), lambda i, j: (0, 0)),
            pl.BlockSpec((_C, 1), lambda i, j: (0, 0)),
        ],
        out_specs=pl.BlockSpec((1, c, bn), lambda i, j: (i, 0, j)),
        out_shape=jax.ShapeDtypeStruct((b, c, n), jnp.float32),
        interpret=interpret,
    )(x, w1, b1c, w2, b2c)
    return out.reshape(b, c, h, w)


def kernel(feat_map0, feat_map1, feat_map2, feat_map3, W1, b1, W2, b2):
    b1c = b1.reshape(_HID, 1)
    b2c = b2.reshape(_C, 1)
    out0 = _mlp_map(feat_map0, W1, b1c, W2, b2c, block_n=1024)
    out1 = _mlp_map(feat_map1, W1, b1c, W2, b2c, block_n=1024)
    return (out0, out1, feat_map2, feat_map3)
